# baseline (speedup 1.0000x reference)
import dataclasses
import functools
import math

import numpy as np

import jax
import jax.numpy as jnp
from jax import lax
from jax.experimental import pallas as pl
from jax.experimental.pallas import tpu as pltpu
from jax.experimental.pallas import tpu_sc as plsc

N = 10000
NP = 10240
E = 160000
D = 128
D_ATTR = 16
D_EMB = 16
D_EDGE = 4
FC_HID = 8

NC = 2
NS = 16
NW = NC * NS
EB = 64
EPAD = 163840
NBLKS = EPAD // EB
C0 = 96
C1 = 64

NODE_BLK = 2000
EDGE_BLK = 1280
IDXR = 20

_LN2 = math.log(2.0)
_HI = lax.Precision.HIGHEST


def _ssp(x):
    return jnp.maximum(x, 0.0) + jnp.log(1.0 + jnp.exp(-jnp.abs(x))) - _LN2


def _dot(a, b):
    return jnp.dot(a, b, preferred_element_type=jnp.float32, precision=_HI)


def _dotd(a, b):
    return jnp.dot(a, b, preferred_element_type=jnp.float32,
                   precision=lax.Precision.DEFAULT)


def _h_body(x_ref, w_ref, o_ref):
    o_ref[...] = _dot(x_ref[...], w_ref[...])


def _prep_body(embt_ref, attrt_ref, src_ref, dst_ref, wr1t_ref, br1_ref,
               wr2_ref, rw_ref, srcp_ref, dstp_ref):
    i = pl.program_id(0)
    embt = embt_ref[...]
    hidt = _ssp(_dotd(wr1t_ref[...], embt) + br1_ref[...])
    ecol = i * EDGE_BLK + lax.broadcasted_iota(jnp.int32, (1, EDGE_BLK), 1)
    attrt = jnp.where(ecol < E, attrt_ref[...], 0.0)
    gt = jnp.concatenate(
        [attrt[v:v + 1, :] * hidt for v in range(D_EDGE)]
        + [attrt, jnp.zeros_like(attrt)],
        axis=0)
    g = jnp.transpose(gt)
    rw_ref[...] = _dotd(g, wr2_ref[...]).astype(jnp.bfloat16)
    irow = i * IDXR + lax.broadcasted_iota(jnp.int32, (1, IDXR, 1), 1)
    ivalid = irow < (E // EB)
    srcp_ref[...] = jnp.where(ivalid, src_ref[...], 0)
    dstp_ref[...] = jnp.where(ivalid, dst_ref[...], 0)


def _scon_body(nf_ref, na_ref, wsc_ref, o_ref):
    nf = nf_ref[...]
    na = na_ref[...]
    acc = na[:, 0:1] * _dotd(nf, wsc_ref[0])
    for j in range(1, D_ATTR):
        acc = acc + na[:, j:j + 1] * _dotd(nf, wsc_ref[j])
    o_ref[...] = acc


def _fin_body(p0_ref, p1_ref, sc_ref, nf_ref, w2_ref, o_ref):
    agg = p0_ref[...] + p1_ref[...]
    lin2 = _dot(agg, w2_ref[...])
    o_ref[...] = _ssp(lin2 + sc_ref[...]) + nf_ref[...]


def _sc_aggregate(h, rw, src2d, dst2d):
    mesh = plsc.VectorSubcoreMesh(core_axis_name="c", subcore_axis_name="s")
    cp = pltpu.CompilerParams()
    if "needs_layout_passes" in pltpu.CompilerParams.__dataclass_fields__:
        cp = dataclasses.replace(cp, needs_layout_passes=False)
    EBP = EB // 2

    @functools.partial(
        pl.kernel,
        out_type=jax.ShapeDtypeStruct((NC * NP, D), jnp.float32),
        mesh=mesh,
        compiler_params=cp,
        scratch_types=[
            pltpu.VMEM((C0 // 4, EB), jnp.int32),
            pltpu.VMEM((C0 // 4, EB), jnp.int32),
            pltpu.VMEM((EB, D), jnp.float32),
            pltpu.VMEM((EB, D), jnp.float32),
            pltpu.VMEM((EBP, D), jnp.float32),
            pltpu.VMEM((EBP, D), jnp.float32),
            pltpu.VMEM((EB, D), jnp.float32),
            pltpu.VMEM((EB, D), jnp.float32),
            pltpu.VMEM_SHARED((NP, D), jnp.float32),
            pltpu.SemaphoreType.DMA,
            pltpu.SemaphoreType.DMA,
            pltpu.SemaphoreType.DMA,
            pltpu.SemaphoreType.DMA,
            pltpu.SemaphoreType.DMA,
            pltpu.SemaphoreType.DMA,
        ],
    )
    def body(h_hbm, rw_hbm, src_hbm, dst_hbm, out_hbm,
             src_v, dst_v, hs0, hs1, rw0, rw1, pf0, pf1, acc,
             g0, g1, r0, r1, s0, s1):
        c = lax.axis_index("c")
        s = lax.axis_index("s")

        @pl.loop(0, EB)
        def _(i):
            for ch in range(D // 16):
                pf0[i, pl.ds(ch * 16, 16)] = jnp.zeros((16,), jnp.float32)

        rows = NP // NS

        @pl.loop(0, rows // EB)
        def _(k):
            pltpu.sync_copy(pf0, acc.at[pl.ds(s * rows + k * EB, EB)])

        plsc.subcore_barrier()

        nblk = jnp.where(c == 0, C0, C1)
        base_blk = jnp.where(c == 0, s * C0, NS * C0 + s * C1)
        hb = nblk // 4

        def start_gather(j, hs, gsem):
            pltpu.async_copy(h_hbm.at[src_v.at[j]], hs, gsem)

        def start_rwfill(b0, j, rw, rsem):
            e0 = pl.multiple_of((b0 + j) * EBP, EBP)
            pltpu.async_copy(rw_hbm.at[pl.ds(e0, EBP)], rw, rsem)

        def wait_gather(j, hs, gsem):
            pltpu.make_async_copy(h_hbm.at[src_v.at[j]], hs, gsem).wait()

        def wait_rwfill(b0, j, rw, rsem):
            e0 = pl.multiple_of((b0 + j) * EBP, EBP)
            pltpu.make_async_copy(rw_hbm.at[pl.ds(e0, EBP)],
                                  rw, rsem).wait()

        def compute(hs, rw, pf):
            @pl.loop(0, EBP)
            def _(k):
                for ch in range(D // 16):
                    rww = rw[k, pl.ds(ch * 16, 16)]
                    ra, rb2 = plsc.unpack(
                        plsc.bitcast(rww, jnp.bfloat16),
                        format=plsc.PackFormat.INTERLEAVED,
                        preferred_element_type=jnp.float32)
                    sl = pl.ds(ch * 16, 16)
                    pf[2 * k, sl] = hs[2 * k, sl] * ra
                    pf[2 * k + 1, sl] = hs[2 * k + 1, sl] * rb2

        def scatter_start(j, pf, ssem):
            pltpu.async_copy(pf, acc.at[dst_v.at[j]], ssem, add=True)

        def scatter_wait(j, pf, ssem):
            pltpu.make_async_copy(pf, acc.at[dst_v.at[j]], ssem).wait()

        for qt in range(4):
            b0 = pl.multiple_of(base_blk + qt * hb, 8)

            @pl.when(c == 0)
            def _():
                pltpu.sync_copy(src_hbm.at[pl.ds(b0, C0 // 4)],
                                src_v.at[pl.ds(0, C0 // 4)])
                pltpu.sync_copy(dst_hbm.at[pl.ds(b0, C0 // 4)],
                                dst_v.at[pl.ds(0, C0 // 4)])

            @pl.when(c == 1)
            def _():
                pltpu.sync_copy(src_hbm.at[pl.ds(b0, C1 // 4)],
                                src_v.at[pl.ds(0, C1 // 4)])
                pltpu.sync_copy(dst_hbm.at[pl.ds(b0, C1 // 4)],
                                dst_v.at[pl.ds(0, C1 // 4)])

            start_gather(0, hs0, g0)
            start_rwfill(b0, 0, rw0, r0)
            start_gather(1, hs1, g1)
            start_rwfill(b0, 1, rw1, r1)

            @pl.loop(0, hb // 2)
            def _(p):
                j0 = 2 * p
                j1 = j0 + 1

                wait_gather(j0, hs0, g0)
                wait_rwfill(b0, j0, rw0, r0)

                @pl.when(p > 0)
                def _():
                    scatter_wait(j0, pf0, s0)

                compute(hs0, rw0, pf0)

                @pl.when(j0 + 2 < hb)
                def _():
                    start_gather(j0 + 2, hs0, g0)
                    start_rwfill(b0, j0 + 2, rw0, r0)

                scatter_start(j0, pf0, s0)

                wait_gather(j1, hs1, g1)
                wait_rwfill(b0, j1, rw1, r1)

                @pl.when(p > 0)
                def _():
                    scatter_wait(j1, pf1, s1)

                compute(hs1, rw1, pf1)

                @pl.when(j1 + 2 < hb)
                def _():
                    start_gather(j1 + 2, hs1, g1)
                    start_rwfill(b0, j1 + 2, rw1, r1)

                scatter_start(j1, pf1, s1)

            scatter_wait(hb - 2, pf0, s0)
            scatter_wait(hb - 1, pf1, s1)

        plsc.subcore_barrier()

        @pl.loop(0, rows // EB)
        def _(k):
            r0 = s * rows + k * EB
            pltpu.sync_copy(acc.at[pl.ds(r0, EB)],
                            out_hbm.at[pl.ds(c * NP + r0, EB)])

    return body(h, rw, src2d, dst2d)


def kernel(node_feats, node_attrs, edge_embedding, edge_attrs, edge_index,
           W1, Wr1, br1, Wr2, br2, W2, Wsc):
    f32 = jnp.float32
    nf = node_feats.astype(f32)
    na = node_attrs.astype(f32)

    inv_se = 1.0 / math.sqrt(float(D_EDGE))
    m1 = (Wr2.reshape(FC_HID, D, D_EDGE).transpose(2, 0, 1)
          .reshape(D_EDGE * FC_HID, D))
    bbm = br2.reshape(D, D_EDGE).T
    wr2r = jnp.concatenate([m1, bbm, jnp.zeros((D_EDGE, D), f32)],
                           axis=0) * inv_se
    br1r = br1.reshape(FC_HID, 1)
    w2s = W2 * (1.0 / math.sqrt(16.0))
    wsct = Wsc.transpose(1, 0, 2) * (1.0 / math.sqrt(float(D * D_ATTR)))

    ei = edge_index.astype(jnp.int32)
    src3 = ei[0].reshape(E // (EB * IDXR), IDXR, EB)
    dst3 = ei[1].reshape(E // (EB * IDXR), IDXR, EB)

    n_nb = N // NODE_BLK

    h = pl.pallas_call(
        _h_body,
        grid=(n_nb,),
        in_specs=[
            pl.BlockSpec((NODE_BLK, D), lambda i: (i, 0)),
            pl.BlockSpec((D, D), lambda i: (0, 0)),
        ],
        out_specs=pl.BlockSpec((NODE_BLK, D), lambda i: (i, 0)),
        out_shape=jax.ShapeDtypeStruct((N, D), f32),
    )(nf, W1)

    g_prep = EPAD // EDGE_BLK
    _clampt = lambda i: (0, jnp.minimum(i, E // EDGE_BLK - 1))
    _clamp3 = lambda i: (jnp.minimum(i, E // EDGE_BLK - 1), 0, 0)
    rw, srcp3, dstp3 = pl.pallas_call(
        _prep_body,
        grid=(g_prep,),
        in_specs=[
            pl.BlockSpec((D_EMB, EDGE_BLK), _clampt),
            pl.BlockSpec((D_EDGE, EDGE_BLK), _clampt),
            pl.BlockSpec((1, IDXR, EB), _clamp3),
            pl.BlockSpec((1, IDXR, EB), _clamp3),
            pl.BlockSpec((FC_HID, D_EMB), lambda i: (0, 0)),
            pl.BlockSpec((FC_HID, 1), lambda i: (0, 0)),
            pl.BlockSpec((5 * FC_HID, D), lambda i: (0, 0)),
        ],
        out_specs=[
            pl.BlockSpec((EDGE_BLK, D), lambda i: (i, 0)),
            pl.BlockSpec((1, IDXR, EB), lambda i: (i, 0, 0)),
            pl.BlockSpec((1, IDXR, EB), lambda i: (i, 0, 0)),
        ],
        out_shape=[
            jax.ShapeDtypeStruct((EPAD, D), jnp.bfloat16),
            jax.ShapeDtypeStruct((g_prep, IDXR, EB), jnp.int32),
            jax.ShapeDtypeStruct((g_prep, IDXR, EB), jnp.int32),
        ],
    )(edge_embedding.astype(f32).T, edge_attrs.astype(f32).T, src3, dst3,
      Wr1.T, br1r, wr2r)
    srcp = srcp3.reshape(NBLKS, EB)
    dstp = dstp3.reshape(NBLKS, EB)
    rwp = lax.bitcast_convert_type(
        rw.reshape(EPAD // 2, 2, D).swapaxes(1, 2), f32)

    partials = _sc_aggregate(h, rwp, srcp, dstp)
    p0 = partials[:N]
    p1 = partials[NP:NP + N]

    scon = pl.pallas_call(
        _scon_body,
        grid=(n_nb,),
        in_specs=[
            pl.BlockSpec((NODE_BLK, D), lambda i: (i, 0)),
            pl.BlockSpec((NODE_BLK, D_ATTR), lambda i: (i, 0)),
            pl.BlockSpec((D_ATTR, D, D), lambda i: (0, 0, 0)),
        ],
        out_specs=pl.BlockSpec((NODE_BLK, D), lambda i: (i, 0)),
        out_shape=jax.ShapeDtypeStruct((N, D), f32),
    )(nf, na, wsct)

    out = pl.pallas_call(
        _fin_body,
        grid=(n_nb,),
        in_specs=[
            pl.BlockSpec((NODE_BLK, D), lambda i: (i, 0)),
            pl.BlockSpec((NODE_BLK, D), lambda i: (i, 0)),
            pl.BlockSpec((NODE_BLK, D), lambda i: (i, 0)),
            pl.BlockSpec((NODE_BLK, D), lambda i: (i, 0)),
            pl.BlockSpec((D, D), lambda i: (0, 0)),
        ],
        out_specs=pl.BlockSpec((NODE_BLK, D), lambda i: (i, 0)),
        out_shape=jax.ShapeDtypeStruct((N, D), f32),
    )(p0, p1, scon, nf, w2s)

    return out


# speedup vs baseline: 2.2108x; 2.2108x over previous
import functools
import math

import jax
import jax.numpy as jnp
from jax import lax
from jax.experimental import pallas as pl
from jax.experimental.pallas import tpu as pltpu
from jax.experimental.pallas import tpu_sc as plsc

N = 10000
NP = 10240
E = 160000
D = 128
D_ATTR = 16
D_EMB = 16
D_EDGE = 4
FC_HID = 8

NC = 2
NS = 16
NW = NC * NS
EB = 64
EPAD = 163840
NBLKS = EPAD // EB
C0 = 112
C1 = 48

NODE_BLK = 2000
EDGE_BLK = 1280
IDXR = 20

_LN2 = math.log(2.0)
_HI = lax.Precision.HIGHEST


def _ssp(x):
    return jnp.maximum(x, 0.0) + jnp.log(1.0 + jnp.exp(-jnp.abs(x))) - _LN2


def _dot(a, b):
    return jnp.dot(a, b, preferred_element_type=jnp.float32, precision=_HI)


def _dotd(a, b):
    return jnp.dot(a, b, preferred_element_type=jnp.float32,
                   precision=lax.Precision.DEFAULT)


def _h_body(x_ref, w_ref, o_ref):
    o_ref[...] = _dot(x_ref[...], w_ref[...])


def _prep_body(embt_ref, attrt_ref, src_ref, dst_ref, wr1t_ref, br1_ref,
               wr2_ref, rw_ref, srcp_ref, dstp_ref):
    i = pl.program_id(0)
    embt = embt_ref[...]
    hidt = _ssp(_dotd(wr1t_ref[...], embt) + br1_ref[...])
    ecol = i * EDGE_BLK + lax.broadcasted_iota(jnp.int32, (1, EDGE_BLK), 1)
    attrt = jnp.where(ecol < E, attrt_ref[...], 0.0)
    gt = jnp.concatenate(
        [attrt[v:v + 1, :] * hidt for v in range(D_EDGE)]
        + [attrt, jnp.zeros_like(attrt)],
        axis=0)
    g = jnp.transpose(gt)
    rw_ref[...] = _dotd(g, wr2_ref[...])
    irow = i * IDXR + lax.broadcasted_iota(jnp.int32, (1, IDXR, 1), 1)
    ivalid = irow < (E // EB)
    srcp_ref[...] = jnp.where(ivalid, src_ref[...], 0)
    dstp_ref[...] = jnp.where(ivalid, dst_ref[...], 0)


def _scon_body(nf_ref, na_ref, wsc_ref, o_ref):
    nf = nf_ref[...]
    na = na_ref[...]
    acc = na[:, 0:1] * _dotd(nf, wsc_ref[0])
    for j in range(1, D_ATTR):
        acc = acc + na[:, j:j + 1] * _dotd(nf, wsc_ref[j])
    o_ref[...] = acc


def _fin_body(p0_ref, p1_ref, sc_ref, nf_ref, w2_ref, o_ref):
    agg = p0_ref[...] + p1_ref[...]
    lin2 = _dot(agg, w2_ref[...])
    o_ref[...] = _ssp(lin2 + sc_ref[...]) + nf_ref[...]


def _sc_aggregate(h, rw, src2d, dst2d):
    mesh = plsc.VectorSubcoreMesh(core_axis_name="c", subcore_axis_name="s")

    @functools.partial(
        pl.kernel,
        out_type=jax.ShapeDtypeStruct((NC * NP, D), jnp.float32),
        mesh=mesh,
        scratch_types=[
            pltpu.VMEM((C0 // 2, EB), jnp.int32),
            pltpu.VMEM((C0 // 2, EB), jnp.int32),
            pltpu.VMEM((EB, D), jnp.float32),
            pltpu.VMEM((EB, D), jnp.float32),
            pltpu.VMEM((EB, D), jnp.float32),
            pltpu.VMEM((EB, D), jnp.float32),
            pltpu.VMEM_SHARED((NP, D), jnp.float32),
            pltpu.SemaphoreType.DMA,
            pltpu.SemaphoreType.DMA,
            pltpu.SemaphoreType.DMA,
            pltpu.SemaphoreType.DMA,
            pltpu.SemaphoreType.DMA,
            pltpu.SemaphoreType.DMA,
        ],
    )
    def body(h_hbm, rw_hbm, src_hbm, dst_hbm, out_hbm,
             src_v, dst_v, hs0, hs1, rw0, rw1, acc,
             g0, g1, r0, r1, s0, s1):
        c = lax.axis_index("c")
        s = lax.axis_index("s")

        @pl.loop(0, EB)
        def _(i):
            for ch in range(D // 16):
                hs0[i, pl.ds(ch * 16, 16)] = jnp.zeros((16,), jnp.float32)

        rows = NP // NS

        @pl.loop(0, rows // EB)
        def _(k):
            pltpu.sync_copy(hs0, acc.at[pl.ds(s * rows + k * EB, EB)])

        plsc.subcore_barrier()

        nblk = jnp.where(c == 0, C0, C1)
        base_blk = jnp.where(c == 0, s * C0, NS * C0 + s * C1)
        hb = nblk // 2

        def start_gather(j, hs, gsem):
            pltpu.async_copy(h_hbm.at[src_v.at[j]], hs, gsem)

        def start_rwfill(b0, j, rw, rsem):
            e0 = pl.multiple_of((b0 + j) * EB, EB)
            pltpu.async_copy(rw_hbm.at[pl.ds(e0, EB)], rw, rsem)

        def wait_gather(j, hs, gsem):
            pltpu.make_async_copy(h_hbm.at[src_v.at[j]], hs, gsem).wait()

        def wait_rwfill(b0, j, rw, rsem):
            e0 = pl.multiple_of((b0 + j) * EB, EB)
            pltpu.make_async_copy(rw_hbm.at[pl.ds(e0, EB)],
                                  rw, rsem).wait()

        def compute(hs, rw):
            @pl.loop(0, EB)
            def _(i):
                for ch in range(D // 16):
                    sl = (i, pl.ds(ch * 16, 16))
                    rw[sl] = rw[sl] * hs[sl]

        def scatter_start(j, rw, ssem):
            pltpu.async_copy(rw, acc.at[dst_v.at[j]], ssem, add=True)

        def scatter_wait(j, rw, ssem):
            pltpu.make_async_copy(rw, acc.at[dst_v.at[j]], ssem).wait()

        for half in range(2):
            b0 = pl.multiple_of(base_blk + half * hb, 8)

            @pl.when(c == 0)
            def _():
                pltpu.sync_copy(src_hbm.at[pl.ds(b0, C0 // 2)],
                                src_v.at[pl.ds(0, C0 // 2)])
                pltpu.sync_copy(dst_hbm.at[pl.ds(b0, C0 // 2)],
                                dst_v.at[pl.ds(0, C0 // 2)])

            @pl.when(c == 1)
            def _():
                pltpu.sync_copy(src_hbm.at[pl.ds(b0, C1 // 2)],
                                src_v.at[pl.ds(0, C1 // 2)])
                pltpu.sync_copy(dst_hbm.at[pl.ds(b0, C1 // 2)],
                                dst_v.at[pl.ds(0, C1 // 2)])

            start_gather(0, hs0, g0)
            start_rwfill(b0, 0, rw0, r0)
            start_gather(1, hs1, g1)
            start_rwfill(b0, 1, rw1, r1)

            @pl.loop(0, hb // 2)
            def _(p):
                j0 = 2 * p
                j1 = j0 + 1

                wait_gather(j0, hs0, g0)
                wait_rwfill(b0, j0, rw0, r0)
                compute(hs0, rw0)

                @pl.when(j0 + 2 < hb)
                def _():
                    start_gather(j0 + 2, hs0, g0)

                scatter_start(j0, rw0, s0)

                wait_gather(j1, hs1, g1)
                wait_rwfill(b0, j1, rw1, r1)
                compute(hs1, rw1)

                @pl.when(j1 + 2 < hb)
                def _():
                    start_gather(j1 + 2, hs1, g1)

                scatter_start(j1, rw1, s1)

                scatter_wait(j0, rw0, s0)

                @pl.when(j0 + 2 < hb)
                def _():
                    start_rwfill(b0, j0 + 2, rw0, r0)

                scatter_wait(j1, rw1, s1)

                @pl.when(j1 + 2 < hb)
                def _():
                    start_rwfill(b0, j1 + 2, rw1, r1)

        plsc.subcore_barrier()

        @pl.loop(0, rows // EB)
        def _(k):
            r0 = s * rows + k * EB
            pltpu.sync_copy(acc.at[pl.ds(r0, EB)],
                            out_hbm.at[pl.ds(c * NP + r0, EB)])

    return body(h, rw, src2d, dst2d)


def kernel(node_feats, node_attrs, edge_embedding, edge_attrs, edge_index,
           W1, Wr1, br1, Wr2, br2, W2, Wsc):
    f32 = jnp.float32
    nf = node_feats.astype(f32)
    na = node_attrs.astype(f32)

    inv_se = 1.0 / math.sqrt(float(D_EDGE))
    m1 = (Wr2.reshape(FC_HID, D, D_EDGE).transpose(2, 0, 1)
          .reshape(D_EDGE * FC_HID, D))
    bbm = br2.reshape(D, D_EDGE).T
    wr2r = jnp.concatenate([m1, bbm, jnp.zeros((D_EDGE, D), f32)],
                           axis=0) * inv_se
    br1r = br1.reshape(FC_HID, 1)
    w2s = W2 * (1.0 / math.sqrt(16.0))
    wsct = Wsc.transpose(1, 0, 2) * (1.0 / math.sqrt(float(D * D_ATTR)))

    ei = edge_index.astype(jnp.int32)
    src3 = ei[0].reshape(E // (EB * IDXR), IDXR, EB)
    dst3 = ei[1].reshape(E // (EB * IDXR), IDXR, EB)

    n_nb = N // NODE_BLK

    h = pl.pallas_call(
        _h_body,
        grid=(n_nb,),
        in_specs=[
            pl.BlockSpec((NODE_BLK, D), lambda i: (i, 0)),
            pl.BlockSpec((D, D), lambda i: (0, 0)),
        ],
        out_specs=pl.BlockSpec((NODE_BLK, D), lambda i: (i, 0)),
        out_shape=jax.ShapeDtypeStruct((N, D), f32),
    )(nf, W1)

    g_prep = EPAD // EDGE_BLK
    _clampt = lambda i: (0, jnp.minimum(i, E // EDGE_BLK - 1))
    _clamp3 = lambda i: (jnp.minimum(i, E // EDGE_BLK - 1), 0, 0)
    rw, srcp3, dstp3 = pl.pallas_call(
        _prep_body,
        grid=(g_prep,),
        in_specs=[
            pl.BlockSpec((D_EMB, EDGE_BLK), _clampt),
            pl.BlockSpec((D_EDGE, EDGE_BLK), _clampt),
            pl.BlockSpec((1, IDXR, EB), _clamp3),
            pl.BlockSpec((1, IDXR, EB), _clamp3),
            pl.BlockSpec((FC_HID, D_EMB), lambda i: (0, 0)),
            pl.BlockSpec((FC_HID, 1), lambda i: (0, 0)),
            pl.BlockSpec((5 * FC_HID, D), lambda i: (0, 0)),
        ],
        out_specs=[
            pl.BlockSpec((EDGE_BLK, D), lambda i: (i, 0)),
            pl.BlockSpec((1, IDXR, EB), lambda i: (i, 0, 0)),
            pl.BlockSpec((1, IDXR, EB), lambda i: (i, 0, 0)),
        ],
        out_shape=[
            jax.ShapeDtypeStruct((EPAD, D), f32),
            jax.ShapeDtypeStruct((g_prep, IDXR, EB), jnp.int32),
            jax.ShapeDtypeStruct((g_prep, IDXR, EB), jnp.int32),
        ],
    )(edge_embedding.astype(f32).T, edge_attrs.astype(f32).T, src3, dst3,
      Wr1.T, br1r, wr2r)
    srcp = srcp3.reshape(NBLKS, EB)
    dstp = dstp3.reshape(NBLKS, EB)

    partials = _sc_aggregate(h, rw, srcp, dstp)
    p0 = partials[:N]
    p1 = partials[NP:NP + N]

    scon = pl.pallas_call(
        _scon_body,
        grid=(n_nb,),
        in_specs=[
            pl.BlockSpec((NODE_BLK, D), lambda i: (i, 0)),
            pl.BlockSpec((NODE_BLK, D_ATTR), lambda i: (i, 0)),
            pl.BlockSpec((D_ATTR, D, D), lambda i: (0, 0, 0)),
        ],
        out_specs=pl.BlockSpec((NODE_BLK, D), lambda i: (i, 0)),
        out_shape=jax.ShapeDtypeStruct((N, D), f32),
    )(nf, na, wsct)

    out = pl.pallas_call(
        _fin_body,
        grid=(n_nb,),
        in_specs=[
            pl.BlockSpec((NODE_BLK, D), lambda i: (i, 0)),
            pl.BlockSpec((NODE_BLK, D), lambda i: (i, 0)),
            pl.BlockSpec((NODE_BLK, D), lambda i: (i, 0)),
            pl.BlockSpec((NODE_BLK, D), lambda i: (i, 0)),
            pl.BlockSpec((D, D), lambda i: (0, 0)),
        ],
        out_specs=pl.BlockSpec((NODE_BLK, D), lambda i: (i, 0)),
        out_shape=jax.ShapeDtypeStruct((N, D), f32),
    )(p0, p1, scon, nf, w2s)

    return out
